# baseline (speedup 1.0000x reference)
import jax
import jax.numpy as jnp
from jax import lax
from jax.experimental import pallas as pl
from jax.experimental.pallas import tpu as pltpu
from jax.experimental.pallas import tpu_sc as plsc

_N = 10000
_K = 32
_D = 128
_LANES = 16
_NUM_CORES = 2
_NUM_SUBCORES = 16
_NUM_WORKERS = _NUM_CORES * _NUM_SUBCORES
_CHUNK_ROWS = 16
_NUM_CHUNKS = _N // _CHUNK_ROWS
_CHUNKS_PER_WORKER = _NUM_CHUNKS // _NUM_WORKERS
_EXTRA_CHUNKS = _NUM_CHUNKS - _CHUNKS_PER_WORKER * _NUM_WORKERS


def _sc_body(nbr, out, acc, obuf, idx, gsem):
    c = lax.axis_index("c")
    s = lax.axis_index("s")
    wid = s * _NUM_CORES + c
    count = _CHUNKS_PER_WORKER + (wid < _EXTRA_CHUNKS).astype(jnp.int32)
    first = _CHUNKS_PER_WORKER * wid + jnp.minimum(wid, _EXTRA_CHUNKS)

    iota32 = lax.iota(jnp.int32, _LANES) * _K
    inv = jnp.full((_LANES,), 1.0 / _K, jnp.float32)
    zero = jnp.zeros((_LANES,), jnp.float32)

    for r in range(_CHUNK_ROWS):
        for j in range(_D // _LANES):
            acc[r, pl.ds(j * _LANES, _LANES)] = zero

    def chunk(g, carry):
        cid = first + g
        base32 = cid * (_CHUNK_ROWS * _K)
        for k in range(_K):
            idx[k] = iota32 + (base32 + k)
        descs = []
        for k in range(_K):
            descs.append(
                pltpu.async_copy(nbr.at[idx.at[k]], acc, gsem, add=True)
            )
        for d in descs:
            d.wait()
        for r in range(_CHUNK_ROWS):
            for j in range(_D // _LANES):
                sl = pl.ds(j * _LANES, _LANES)
                obuf[r, sl] = acc[r, sl] * inv
                acc[r, sl] = zero
        pltpu.sync_copy(obuf, out.at[pl.ds(cid * _CHUNK_ROWS, _CHUNK_ROWS)])
        return carry

    lax.fori_loop(0, count, chunk, 0)


def kernel(neighbor):
    n, k, d = neighbor.shape
    rows = neighbor.reshape(n * k, d)
    mesh = plsc.VectorSubcoreMesh(
        core_axis_name="c",
        subcore_axis_name="s",
        num_cores=_NUM_CORES,
        num_subcores=_NUM_SUBCORES,
    )
    run = pl.kernel(
        _sc_body,
        out_type=jax.ShapeDtypeStruct((n, d), neighbor.dtype),
        mesh=mesh,
        scratch_types=[
            pltpu.VMEM((_CHUNK_ROWS, _D), jnp.float32),
            pltpu.VMEM((_CHUNK_ROWS, _D), jnp.float32),
            pltpu.VMEM((_K, _LANES), jnp.int32),
            pltpu.SemaphoreType.DMA,
        ],
    )
    return run(rows)
